# baseline (speedup 1.0000x reference)
import jax
import jax.numpy as jnp
from jax.experimental import pallas as pl

_BLK = 1024


def _fused_body(x_ref, wz_ref, wh_ref, bz_ref, bh_ref, wl_ref, bl_ref, o_ref):
    xb = x_ref[:]
    z = jax.nn.sigmoid(
        jnp.dot(xb, wz_ref[:], preferred_element_type=jnp.float32) + bz_ref[:]
    )
    t = jnp.tanh(
        jnp.dot(xb, wh_ref[:], preferred_element_type=jnp.float32) + bh_ref[:]
    )
    h = jnp.maximum((1.0 - z) * t, 0.0)
    o_ref[:] = (
        jnp.dot(h, wl_ref[:], preferred_element_type=jnp.float32) + bl_ref[:]
    )


def kernel(x, edge_index, edge_weight, W_xz, b_xz, W_hz, b_hz, W_xr, b_xr,
           W_hr, b_hr, W_xh, b_xh, W_hh, b_hh, W_lin, b_lin):
    n, f = x.shape
    hid = W_xz.shape[1]
    bz = (b_xz + b_hz).reshape(1, hid)
    bh = (b_xh + b_hh).reshape(1, hid)
    wl = W_lin
    bl = b_lin.reshape(1, 1)
    out = pl.pallas_call(
        _fused_body,
        grid=(pl.cdiv(n, _BLK),),
        in_specs=[
            pl.BlockSpec((_BLK, f), lambda i: (i, 0)),
            pl.BlockSpec((f, hid), lambda i: (0, 0)),
            pl.BlockSpec((f, hid), lambda i: (0, 0)),
            pl.BlockSpec((1, hid), lambda i: (0, 0)),
            pl.BlockSpec((1, hid), lambda i: (0, 0)),
            pl.BlockSpec((hid, 1), lambda i: (0, 0)),
            pl.BlockSpec((1, 1), lambda i: (0, 0)),
        ],
        out_specs=pl.BlockSpec((_BLK, 1), lambda i: (i, 0)),
        out_shape=jax.ShapeDtypeStruct((n, 1), jnp.float32),
    )(x, W_xz, W_xh, bz, bh, wl, bl)
    return out


# speedup vs baseline: 1.0643x; 1.0643x over previous
import jax
import jax.numpy as jnp
from jax.experimental import pallas as pl

_BLK = 1024


def _fused_body(x_ref, wz_ref, wh_ref, bxz_ref, bhz_ref, bxh_ref, bhh_ref,
                wl_ref, bl_ref, o_ref):
    xb = x_ref[:]
    z = jax.nn.sigmoid(
        jnp.dot(xb, wz_ref[:], preferred_element_type=jnp.float32)
        + (bxz_ref[:] + bhz_ref[:])
    )
    t = jnp.tanh(
        jnp.dot(xb, wh_ref[:], preferred_element_type=jnp.float32)
        + (bxh_ref[:] + bhh_ref[:])
    )
    h = jnp.maximum((1.0 - z) * t, 0.0)
    o_ref[:] = (
        jnp.dot(h, wl_ref[:], preferred_element_type=jnp.float32) + bl_ref[:]
    )


def kernel(x, edge_index, edge_weight, W_xz, b_xz, W_hz, b_hz, W_xr, b_xr,
           W_hr, b_hr, W_xh, b_xh, W_hh, b_hh, W_lin, b_lin):
    n, f = x.shape
    hid = W_xz.shape[1]
    _vec = pl.BlockSpec((1, hid), lambda i: (0, 0))
    out = pl.pallas_call(
        _fused_body,
        grid=(pl.cdiv(n, _BLK),),
        in_specs=[
            pl.BlockSpec((_BLK, f), lambda i: (i, 0)),
            pl.BlockSpec((f, hid), lambda i: (0, 0)),
            pl.BlockSpec((f, hid), lambda i: (0, 0)),
            _vec, _vec, _vec, _vec,
            pl.BlockSpec((hid, 1), lambda i: (0, 0)),
            pl.BlockSpec((1, 1), lambda i: (0, 0)),
        ],
        out_specs=pl.BlockSpec((_BLK, 1), lambda i: (i, 0)),
        out_shape=jax.ShapeDtypeStruct((n, 1), jnp.float32),
    )(x, W_xz, W_xh, b_xz.reshape(1, hid), b_hz.reshape(1, hid),
      b_xh.reshape(1, hid), b_hh.reshape(1, hid), W_lin, b_lin.reshape(1, 1))
    return out


# speedup vs baseline: 1.2264x; 1.1522x over previous
import jax
import jax.numpy as jnp
from jax.experimental import pallas as pl

_BLK = 2048


def _fused_body(x_ref, wz_ref, wh_ref, bxz_ref, bhz_ref, bxh_ref, bhh_ref,
                wl_ref, bl_ref, o_ref):
    xb = x_ref[:]
    z = jax.nn.sigmoid(
        jnp.dot(xb, wz_ref[:], preferred_element_type=jnp.float32)
        + (bxz_ref[:] + bhz_ref[:])
    )
    t = jnp.tanh(
        jnp.dot(xb, wh_ref[:], preferred_element_type=jnp.float32)
        + (bxh_ref[:] + bhh_ref[:])
    )
    h = jnp.maximum((1.0 - z) * t, 0.0)
    o_ref[:] = (
        jnp.dot(h, wl_ref[:], preferred_element_type=jnp.float32) + bl_ref[:]
    )


def kernel(x, edge_index, edge_weight, W_xz, b_xz, W_hz, b_hz, W_xr, b_xr,
           W_hr, b_hr, W_xh, b_xh, W_hh, b_hh, W_lin, b_lin):
    n, f = x.shape
    hid = W_xz.shape[1]
    _vec = pl.BlockSpec((1, hid), lambda i: (0, 0))
    out = pl.pallas_call(
        _fused_body,
        grid=(pl.cdiv(n, _BLK),),
        in_specs=[
            pl.BlockSpec((_BLK, f), lambda i: (i, 0)),
            pl.BlockSpec((f, hid), lambda i: (0, 0)),
            pl.BlockSpec((f, hid), lambda i: (0, 0)),
            _vec, _vec, _vec, _vec,
            pl.BlockSpec((hid, 1), lambda i: (0, 0)),
            pl.BlockSpec((1, 1), lambda i: (0, 0)),
        ],
        out_specs=pl.BlockSpec((_BLK, 1), lambda i: (i, 0)),
        out_shape=jax.ShapeDtypeStruct((n, 1), jnp.float32),
    )(x, W_xz, W_xh, b_xz.reshape(1, hid), b_hz.reshape(1, hid),
      b_xh.reshape(1, hid), b_hh.reshape(1, hid), W_lin, b_lin.reshape(1, 1))
    return out
